# Initial kernel scaffold; baseline (speedup 1.0000x reference)
import jax
import jax.numpy as jnp
from jax.experimental import pallas as pl


def kernel(points, features, W0, W1, W2, Wsc, g0, b0, g1, b1, g2, b2, gsc, bsc):
    raise NotImplementedError("write your pallas kernel here")


# speedup vs baseline: 7.2084x; 7.2084x over previous
import functools

import jax
import jax.numpy as jnp
from jax.experimental import pallas as pl
from jax.experimental.pallas import tpu as pltpu

_K = 16
_EPS = 1e-3
_TN = 256


def _edgeconv_tile(pts_ref, ptsT_ref, featsC_ref, fhi_ref, flo_ref,
                   w0a_ref, w0b_ref, w1_ref, w2_ref, wsc_ref,
                   g0_ref, b0_ref, g1_ref, b1_ref, g2_ref, b2_ref,
                   gsc_ref, bsc_ref, out_ref, *, n, k):
    f32 = jnp.float32
    bf16 = jnp.bfloat16
    tn = pts_ref.shape[1]
    sq = jnp.sqrt(f32(1.0 + _EPS))

    pts = pts_ref[0]
    ptsT = ptsT_ref[0]
    fc = featsC_ref[0]
    fhi = fhi_ref[0]
    flo = flo_ref[0]

    pb = pts.astype(bf16).astype(f32)
    tb = ptsT.astype(bf16).astype(f32)
    m = pb[:, 0:1] * tb[0:1, :]
    m = m + pb[:, 1:2] * tb[1:2, :]
    m = m + pb[:, 2:3] * tb[2:3, :]
    r_t = pts[:, 0:1] * pts[:, 0:1]
    r_t = r_t + pts[:, 1:2] * pts[:, 1:2]
    r_t = r_t + pts[:, 2:3] * pts[:, 2:3]
    r_a = ptsT[0:1, :] * ptsT[0:1, :]
    r_a = r_a + ptsT[1:2, :] * ptsT[1:2, :]
    r_a = r_a + ptsT[2:3, :] * ptsT[2:3, :]
    running = (r_t - 2.0 * m) + r_a

    iota = jax.lax.broadcasted_iota(jnp.int32, (tn, n), 1)
    big = jnp.int32(n)

    w0a = w0a_ref[...].astype(bf16)
    w0b = w0b_ref[...].astype(bf16)
    w1 = w1_ref[...].astype(bf16)
    w2 = w2_ref[...].astype(bf16)
    fcb = fc.astype(bf16)

    cpart = jnp.dot(fcb, w0a, preferred_element_type=f32)
    acc = jnp.zeros((tn, w2_ref.shape[1]), f32)

    for step in range(k + 1):
        rowmin = jnp.min(running, axis=1, keepdims=True)
        is_min = running == rowmin
        argmin = jnp.min(jnp.where(is_min, iota, big), axis=1, keepdims=True)
        chosen = iota == argmin
        running = jnp.where(chosen, jnp.inf, running)
        if step == 0:
            continue
        oh = chosen.astype(bf16)
        knn = (jnp.dot(oh, fhi, preferred_element_type=f32)
               + jnp.dot(oh, flo, preferred_element_type=f32))
        diff = (knn - fc).astype(bf16)
        y0 = cpart + jnp.dot(diff, w0b, preferred_element_type=f32)
        h0 = jax.nn.relu(g0_ref[...] * y0 / sq + b0_ref[...])
        y1 = jnp.dot(h0.astype(bf16), w1, preferred_element_type=f32)
        h1 = jax.nn.relu(g1_ref[...] * y1 / sq + b1_ref[...])
        y2 = jnp.dot(h1.astype(bf16), w2, preferred_element_type=f32)
        h2 = jax.nn.relu(g2_ref[...] * y2 / sq + b2_ref[...])
        acc = acc + h2

    fts = acc * f32(1.0 / k)
    ysc = jnp.dot(fcb, wsc_ref[...].astype(bf16), preferred_element_type=f32)
    sc = gsc_ref[...] * ysc / sq + bsc_ref[...]
    out_ref[0] = jax.nn.relu(sc + fts)


def kernel(points, features, W0, W1, W2, Wsc, g0, b0, g1, b1, g2, b2, gsc, bsc):
    b, n, c = features.shape
    tn = _TN
    grid = (b, n // tn)

    pointsT = jnp.transpose(points, (0, 2, 1))
    fhi = features.astype(jnp.bfloat16)
    flo = (features - fhi.astype(jnp.float32)).astype(jnp.bfloat16)
    w0a, w0b = W0[:c], W0[c:]
    row = lambda v: v.reshape(1, -1)

    body = functools.partial(_edgeconv_tile, n=n, k=_K)
    out = pl.pallas_call(
        body,
        grid=grid,
        in_specs=[
            pl.BlockSpec((1, tn, points.shape[2]), lambda bi, ti: (bi, ti, 0)),
            pl.BlockSpec((1, points.shape[2], n), lambda bi, ti: (bi, 0, 0)),
            pl.BlockSpec((1, tn, c), lambda bi, ti: (bi, ti, 0)),
            pl.BlockSpec((1, n, c), lambda bi, ti: (bi, 0, 0)),
            pl.BlockSpec((1, n, c), lambda bi, ti: (bi, 0, 0)),
            pl.BlockSpec(w0a.shape, lambda bi, ti: (0, 0)),
            pl.BlockSpec(w0b.shape, lambda bi, ti: (0, 0)),
            pl.BlockSpec(W1.shape, lambda bi, ti: (0, 0)),
            pl.BlockSpec(W2.shape, lambda bi, ti: (0, 0)),
            pl.BlockSpec(Wsc.shape, lambda bi, ti: (0, 0)),
        ] + [pl.BlockSpec((1, c), lambda bi, ti: (0, 0))] * 8,
        out_specs=pl.BlockSpec((1, tn, c), lambda bi, ti: (bi, ti, 0)),
        out_shape=jax.ShapeDtypeStruct((b, n, c), jnp.float32),
        compiler_params=pltpu.CompilerParams(
            dimension_semantics=("parallel", "parallel")),
    )(points, pointsT, features, fhi, flo, w0a, w0b, W1, W2, Wsc,
      row(g0), row(b0), row(g1), row(b1), row(g2), row(b2), row(gsc), row(bsc))
    return out


# speedup vs baseline: 10.8650x; 1.5073x over previous
import functools

import jax
import jax.numpy as jnp
from jax.experimental import pallas as pl
from jax.experimental.pallas import tpu as pltpu

_K = 16
_EPS = 1e-3
_TN = 256


def _edgeconv_tile(pts_ref, ptsT_ref, featsC_ref, fcat_ref,
                   w0a_ref, bd0_ref, bd1_ref, bd2_ref, wsc_ref,
                   g0_ref, b0_ref, g1_ref, b1_ref, g2_ref, b2_ref,
                   gsc_ref, bsc_ref, out_ref, diff_ref, *, n, k):
    f32 = jnp.float32
    bf16 = jnp.bfloat16
    tn = pts_ref.shape[1]
    c = featsC_ref.shape[2]
    sq = jnp.sqrt(f32(1.0 + _EPS))

    pts = pts_ref[0]
    ptsT = ptsT_ref[0]
    fc = featsC_ref[0]
    fcat = fcat_ref[0]

    pb = pts.astype(bf16).astype(f32)
    tb = ptsT.astype(bf16).astype(f32)
    m = pb[:, 0:1] * tb[0:1, :]
    m = m + pb[:, 1:2] * tb[1:2, :]
    m = m + pb[:, 2:3] * tb[2:3, :]
    r_t = pts[:, 0:1] * pts[:, 0:1]
    r_t = r_t + pts[:, 1:2] * pts[:, 1:2]
    r_t = r_t + pts[:, 2:3] * pts[:, 2:3]
    r_a = ptsT[0:1, :] * ptsT[0:1, :]
    r_a = r_a + ptsT[1:2, :] * ptsT[1:2, :]
    r_a = r_a + ptsT[2:3, :] * ptsT[2:3, :]
    running = (r_t - 2.0 * m) + r_a

    iota = jax.lax.broadcasted_iota(jnp.int32, (tn, n), 1)
    big = jnp.int32(n)

    fcb = fc.astype(bf16)
    cpart = jnp.dot(fcb, w0a_ref[...].astype(bf16),
                    preferred_element_type=f32)

    for step in range(k + 1):
        rowmin = jnp.min(running, axis=1, keepdims=True)
        is_min = running == rowmin
        argmin = jnp.min(jnp.where(is_min, iota, big), axis=1, keepdims=True)
        chosen = iota == argmin
        running = jnp.where(chosen, jnp.inf, running)
        if step == 0:
            continue
        oh = chosen.astype(bf16)
        gh = jnp.dot(oh, fcat, preferred_element_type=f32)
        knn = gh[:, :c] + gh[:, c:]
        diff_ref[:, (step - 1) * c:step * c] = (knn - fc).astype(bf16)

    gg = 4 * c
    cpart4 = jnp.concatenate([cpart] * 4, axis=1)
    t4 = lambda ref: jnp.concatenate([ref[...]] * 4, axis=1)
    g0r, b0r = t4(g0_ref), t4(b0_ref)
    g1r, b1r = t4(g1_ref), t4(b1_ref)
    g2r, b2r = t4(g2_ref), t4(b2_ref)
    acc = jnp.zeros((tn, c), f32)
    for g in range(k // 4):
        xg = diff_ref[:, g * gg:(g + 1) * gg]
        y0 = cpart4 + jnp.dot(xg, bd0_ref[...], preferred_element_type=f32)
        h0 = jax.nn.relu(g0r * y0 / sq + b0r)
        y1 = jnp.dot(h0.astype(bf16), bd1_ref[...], preferred_element_type=f32)
        h1 = jax.nn.relu(g1r * y1 / sq + b1r)
        y2 = jnp.dot(h1.astype(bf16), bd2_ref[...], preferred_element_type=f32)
        h2 = jax.nn.relu(g2r * y2 / sq + b2r)
        acc = acc + ((h2[:, :c] + h2[:, c:2 * c])
                     + (h2[:, 2 * c:3 * c] + h2[:, 3 * c:]))

    fts = acc * f32(1.0 / k)
    ysc = jnp.dot(fcb, wsc_ref[...].astype(bf16), preferred_element_type=f32)
    sc = gsc_ref[...] * ysc / sq + bsc_ref[...]
    out_ref[0] = jax.nn.relu(sc + fts)


def kernel(points, features, W0, W1, W2, Wsc, g0, b0, g1, b1, g2, b2, gsc, bsc):
    b, n, c = features.shape
    tn = _TN
    grid = (b, n // tn)

    pointsT = jnp.transpose(points, (0, 2, 1))
    fhi = features.astype(jnp.bfloat16)
    flo = (features - fhi.astype(jnp.float32)).astype(jnp.bfloat16)
    fcat = jnp.concatenate([fhi, flo], axis=2)
    w0a, w0b = W0[:c], W0[c:]
    eye4 = jnp.eye(4, dtype=jnp.float32)
    bd = lambda w: jnp.kron(eye4, w).astype(jnp.bfloat16)
    row = lambda v: v.reshape(1, -1)

    body = functools.partial(_edgeconv_tile, n=n, k=_K)
    out = pl.pallas_call(
        body,
        grid=grid,
        in_specs=[
            pl.BlockSpec((1, tn, points.shape[2]), lambda bi, ti: (bi, ti, 0)),
            pl.BlockSpec((1, points.shape[2], n), lambda bi, ti: (bi, 0, 0)),
            pl.BlockSpec((1, tn, c), lambda bi, ti: (bi, ti, 0)),
            pl.BlockSpec((1, n, 2 * c), lambda bi, ti: (bi, 0, 0)),
            pl.BlockSpec(w0a.shape, lambda bi, ti: (0, 0)),
            pl.BlockSpec((4 * c, 4 * c), lambda bi, ti: (0, 0)),
            pl.BlockSpec((4 * c, 4 * c), lambda bi, ti: (0, 0)),
            pl.BlockSpec((4 * c, 4 * c), lambda bi, ti: (0, 0)),
            pl.BlockSpec(Wsc.shape, lambda bi, ti: (0, 0)),
        ] + [pl.BlockSpec((1, c), lambda bi, ti: (0, 0))] * 8,
        out_specs=pl.BlockSpec((1, tn, c), lambda bi, ti: (bi, ti, 0)),
        out_shape=jax.ShapeDtypeStruct((b, n, c), jnp.float32),
        scratch_shapes=[pltpu.VMEM((tn, _K * c), jnp.bfloat16)],
        compiler_params=pltpu.CompilerParams(
            dimension_semantics=("parallel", "parallel")),
    )(points, pointsT, features, fcat, w0a, bd(w0b), bd(W1), bd(W2), Wsc,
      row(g0), row(b0), row(g1), row(b1), row(g2), row(b2), row(gsc), row(bsc))
    return out


# speedup vs baseline: 14.3848x; 1.3240x over previous
import functools

import jax
import jax.numpy as jnp
from jax.experimental import pallas as pl
from jax.experimental.pallas import tpu as pltpu

_K = 16
_EPS = 1e-3
_TN = 256


def _edgeconv_tile(pts_ref, ptsT_ref, featsC_ref, fcat_ref,
                   w0a_ref, bd0_ref, bd1_ref, bd2_ref, wsc_ref,
                   g0_ref, b0_ref, g1_ref, b1_ref, g2_ref, b2_ref,
                   gsc_ref, bsc_ref, out_ref, diff_ref, *, n, k):
    f32 = jnp.float32
    bf16 = jnp.bfloat16
    tn = pts_ref.shape[1]
    c = featsC_ref.shape[2]
    sq = jnp.sqrt(f32(1.0 + _EPS))

    pts = pts_ref[0]
    ptsT = ptsT_ref[0]
    fc = featsC_ref[0]
    fcat = fcat_ref[0]

    pb = pts.astype(bf16).astype(f32)
    tb = ptsT.astype(bf16).astype(f32)
    m = pb[:, 0:1] * tb[0:1, :]
    m = m + pb[:, 1:2] * tb[1:2, :]
    m = m + pb[:, 2:3] * tb[2:3, :]
    r_t = pts[:, 0:1] * pts[:, 0:1]
    r_t = r_t + pts[:, 1:2] * pts[:, 1:2]
    r_t = r_t + pts[:, 2:3] * pts[:, 2:3]
    r_a = ptsT[0:1, :] * ptsT[0:1, :]
    r_a = r_a + ptsT[1:2, :] * ptsT[1:2, :]
    r_a = r_a + ptsT[2:3, :] * ptsT[2:3, :]
    dmat = (r_t - 2.0 * m) + r_a

    ib = jax.lax.bitcast_convert_type(dmat, jnp.int32)
    running = jnp.where(ib >= 0, ib, ib ^ jnp.int32(0x7FFFFFFF))
    kmax = jnp.int32(0x7FFFFFFF)

    fcb = fc.astype(bf16)
    cpart = jnp.dot(fcb, w0a_ref[...].astype(bf16),
                    preferred_element_type=f32)

    for step in range(k + 1):
        rowmin = jnp.min(running, axis=1, keepdims=True)
        chosen = running == rowmin
        running = jnp.where(chosen, kmax, running)
        if step == 0:
            continue
        oh = chosen.astype(bf16)
        gh = jnp.dot(oh, fcat, preferred_element_type=f32)
        knn = gh[:, :c] + gh[:, c:]
        diff_ref[:, (step - 1) * c:step * c] = (knn - fc).astype(bf16)

    gg = 4 * c
    cpart4 = jnp.concatenate([cpart] * 4, axis=1)
    t4 = lambda ref: jnp.concatenate([ref[...]] * 4, axis=1)
    g0r, b0r = t4(g0_ref), t4(b0_ref)
    g1r, b1r = t4(g1_ref), t4(b1_ref)
    g2r, b2r = t4(g2_ref), t4(b2_ref)
    acc = jnp.zeros((tn, c), f32)
    for g in range(k // 4):
        xg = diff_ref[:, g * gg:(g + 1) * gg]
        y0 = cpart4 + jnp.dot(xg, bd0_ref[...], preferred_element_type=f32)
        h0 = jax.nn.relu(g0r * y0 / sq + b0r)
        y1 = jnp.dot(h0.astype(bf16), bd1_ref[...], preferred_element_type=f32)
        h1 = jax.nn.relu(g1r * y1 / sq + b1r)
        y2 = jnp.dot(h1.astype(bf16), bd2_ref[...], preferred_element_type=f32)
        h2 = jax.nn.relu(g2r * y2 / sq + b2r)
        acc = acc + ((h2[:, :c] + h2[:, c:2 * c])
                     + (h2[:, 2 * c:3 * c] + h2[:, 3 * c:]))

    fts = acc * f32(1.0 / k)
    ysc = jnp.dot(fcb, wsc_ref[...].astype(bf16), preferred_element_type=f32)
    sc = gsc_ref[...] * ysc / sq + bsc_ref[...]
    out_ref[0] = jax.nn.relu(sc + fts)


def kernel(points, features, W0, W1, W2, Wsc, g0, b0, g1, b1, g2, b2, gsc, bsc):
    b, n, c = features.shape
    tn = _TN
    grid = (b, n // tn)

    pointsT = jnp.transpose(points, (0, 2, 1))
    fhi = features.astype(jnp.bfloat16)
    flo = (features - fhi.astype(jnp.float32)).astype(jnp.bfloat16)
    fcat = jnp.concatenate([fhi, flo], axis=2)
    w0a, w0b = W0[:c], W0[c:]
    eye4 = jnp.eye(4, dtype=jnp.float32)
    bd = lambda w: jnp.kron(eye4, w).astype(jnp.bfloat16)
    row = lambda v: v.reshape(1, -1)

    body = functools.partial(_edgeconv_tile, n=n, k=_K)
    out = pl.pallas_call(
        body,
        grid=grid,
        in_specs=[
            pl.BlockSpec((1, tn, points.shape[2]), lambda bi, ti: (bi, ti, 0)),
            pl.BlockSpec((1, points.shape[2], n), lambda bi, ti: (bi, 0, 0)),
            pl.BlockSpec((1, tn, c), lambda bi, ti: (bi, ti, 0)),
            pl.BlockSpec((1, n, 2 * c), lambda bi, ti: (bi, 0, 0)),
            pl.BlockSpec(w0a.shape, lambda bi, ti: (0, 0)),
            pl.BlockSpec((4 * c, 4 * c), lambda bi, ti: (0, 0)),
            pl.BlockSpec((4 * c, 4 * c), lambda bi, ti: (0, 0)),
            pl.BlockSpec((4 * c, 4 * c), lambda bi, ti: (0, 0)),
            pl.BlockSpec(Wsc.shape, lambda bi, ti: (0, 0)),
        ] + [pl.BlockSpec((1, c), lambda bi, ti: (0, 0))] * 8,
        out_specs=pl.BlockSpec((1, tn, c), lambda bi, ti: (bi, ti, 0)),
        out_shape=jax.ShapeDtypeStruct((b, n, c), jnp.float32),
        scratch_shapes=[pltpu.VMEM((tn, _K * c), jnp.bfloat16)],
        compiler_params=pltpu.CompilerParams(
            dimension_semantics=("parallel", "parallel")),
    )(points, pointsT, features, fcat, w0a, bd(w0b), bd(W1), bd(W2), Wsc,
      row(g0), row(b0), row(g1), row(b1), row(g2), row(b2), row(gsc), row(bsc))
    return out


# speedup vs baseline: 15.9148x; 1.1064x over previous
import functools

import jax
import jax.numpy as jnp
from jax.experimental import pallas as pl
from jax.experimental.pallas import tpu as pltpu

_K = 16
_EPS = 1e-3
_TN = 512


def _edgeconv_tile(pts_ref, ptsT_ref, featsC_ref, fcat_ref,
                   w0a_ref, bd0_ref, bd1_ref, bd2_ref, wsc_ref,
                   g0_ref, b0_ref, g1_ref, b1_ref, g2_ref, b2_ref,
                   gsc_ref, bsc_ref, out_ref, diff_ref, *, n, k):
    f32 = jnp.float32
    bf16 = jnp.bfloat16
    tn = pts_ref.shape[1]
    c = featsC_ref.shape[2]
    sq = jnp.sqrt(f32(1.0 + _EPS))

    pts = pts_ref[0]
    ptsT = ptsT_ref[0]
    fc = featsC_ref[0]
    fcat = fcat_ref[0]

    pb = pts.astype(bf16).astype(f32)
    tb = ptsT.astype(bf16).astype(f32)
    m = pb[:, 0:1] * tb[0:1, :]
    m = m + pb[:, 1:2] * tb[1:2, :]
    m = m + pb[:, 2:3] * tb[2:3, :]
    r_t = pts[:, 0:1] * pts[:, 0:1]
    r_t = r_t + pts[:, 1:2] * pts[:, 1:2]
    r_t = r_t + pts[:, 2:3] * pts[:, 2:3]
    r_a = ptsT[0:1, :] * ptsT[0:1, :]
    r_a = r_a + ptsT[1:2, :] * ptsT[1:2, :]
    r_a = r_a + ptsT[2:3, :] * ptsT[2:3, :]
    dmat = (r_t - 2.0 * m) + r_a

    ib = jax.lax.bitcast_convert_type(dmat, jnp.int32)
    running = jnp.where(ib >= 0, ib, ib ^ jnp.int32(0x7FFFFFFF))
    kmax = jnp.int32(0x7FFFFFFF)

    fcb = fc.astype(bf16)
    cpart = jnp.dot(fcb, w0a_ref[...].astype(bf16),
                    preferred_element_type=f32)

    for step in range(k + 1):
        rowmin = jnp.min(running, axis=1, keepdims=True)
        chosen = running == rowmin
        running = jnp.where(chosen, kmax, running)
        if step == 0:
            continue
        oh = chosen.astype(bf16)
        gh = jnp.dot(oh, fcat, preferred_element_type=f32)
        knn = gh[:, :c] + gh[:, c:]
        diff_ref[:, (step - 1) * c:step * c] = (knn - fc).astype(bf16)

    gg = 4 * c
    cpart4 = jnp.concatenate([cpart] * 4, axis=1)
    t4 = lambda ref: jnp.concatenate([ref[...]] * 4, axis=1)
    g0r, b0r = t4(g0_ref), t4(b0_ref)
    g1r, b1r = t4(g1_ref), t4(b1_ref)
    g2r, b2r = t4(g2_ref), t4(b2_ref)
    acc = jnp.zeros((tn, c), f32)
    for g in range(k // 4):
        xg = diff_ref[:, g * gg:(g + 1) * gg]
        y0 = cpart4 + jnp.dot(xg, bd0_ref[...], preferred_element_type=f32)
        h0 = jax.nn.relu(g0r * y0 / sq + b0r)
        y1 = jnp.dot(h0.astype(bf16), bd1_ref[...], preferred_element_type=f32)
        h1 = jax.nn.relu(g1r * y1 / sq + b1r)
        y2 = jnp.dot(h1.astype(bf16), bd2_ref[...], preferred_element_type=f32)
        h2 = jax.nn.relu(g2r * y2 / sq + b2r)
        acc = acc + ((h2[:, :c] + h2[:, c:2 * c])
                     + (h2[:, 2 * c:3 * c] + h2[:, 3 * c:]))

    fts = acc * f32(1.0 / k)
    ysc = jnp.dot(fcb, wsc_ref[...].astype(bf16), preferred_element_type=f32)
    sc = gsc_ref[...] * ysc / sq + bsc_ref[...]
    out_ref[0] = jax.nn.relu(sc + fts)


def kernel(points, features, W0, W1, W2, Wsc, g0, b0, g1, b1, g2, b2, gsc, bsc):
    b, n, c = features.shape
    tn = _TN
    grid = (b, n // tn)

    pointsT = jnp.transpose(points, (0, 2, 1))
    fhi = features.astype(jnp.bfloat16)
    flo = (features - fhi.astype(jnp.float32)).astype(jnp.bfloat16)
    fcat = jnp.concatenate([fhi, flo], axis=2)
    w0a, w0b = W0[:c], W0[c:]
    eye4 = jnp.eye(4, dtype=jnp.float32)
    bd = lambda w: jnp.kron(eye4, w).astype(jnp.bfloat16)
    row = lambda v: v.reshape(1, -1)

    body = functools.partial(_edgeconv_tile, n=n, k=_K)
    out = pl.pallas_call(
        body,
        grid=grid,
        in_specs=[
            pl.BlockSpec((1, tn, points.shape[2]), lambda bi, ti: (bi, ti, 0)),
            pl.BlockSpec((1, points.shape[2], n), lambda bi, ti: (bi, 0, 0)),
            pl.BlockSpec((1, tn, c), lambda bi, ti: (bi, ti, 0)),
            pl.BlockSpec((1, n, 2 * c), lambda bi, ti: (bi, 0, 0)),
            pl.BlockSpec(w0a.shape, lambda bi, ti: (0, 0)),
            pl.BlockSpec((4 * c, 4 * c), lambda bi, ti: (0, 0)),
            pl.BlockSpec((4 * c, 4 * c), lambda bi, ti: (0, 0)),
            pl.BlockSpec((4 * c, 4 * c), lambda bi, ti: (0, 0)),
            pl.BlockSpec(Wsc.shape, lambda bi, ti: (0, 0)),
        ] + [pl.BlockSpec((1, c), lambda bi, ti: (0, 0))] * 8,
        out_specs=pl.BlockSpec((1, tn, c), lambda bi, ti: (bi, ti, 0)),
        out_shape=jax.ShapeDtypeStruct((b, n, c), jnp.float32),
        scratch_shapes=[pltpu.VMEM((tn, _K * c), jnp.bfloat16)],
        compiler_params=pltpu.CompilerParams(
            dimension_semantics=("parallel", "parallel")),
    )(points, pointsT, features, fcat, w0a, bd(w0b), bd(W1), bd(W2), Wsc,
      row(g0), row(b0), row(g1), row(b1), row(g2), row(b2), row(gsc), row(bsc))
    return out


# speedup vs baseline: 16.9781x; 1.0668x over previous
import functools

import jax
import jax.numpy as jnp
from jax.experimental import pallas as pl
from jax.experimental.pallas import tpu as pltpu

_K = 16
_EPS = 1e-3
_TN = 512


def _edgeconv_tile(pts_ref, ptsT_ref, featsC_ref, fcat_ref,
                   w0a_ref, bd0_ref, bd1_ref, bd2_ref, wsc_ref,
                   g0_ref, b0_ref, g1_ref, b1_ref, g2_ref, b2_ref,
                   gsc_ref, bsc_ref, out_ref, diff_ref, *, n, k):
    f32 = jnp.float32
    bf16 = jnp.bfloat16
    tn = pts_ref.shape[1]
    c = featsC_ref.shape[2]
    sq = jnp.sqrt(f32(1.0 + _EPS))

    pts = pts_ref[0]
    ptsT = ptsT_ref[0]
    fc = featsC_ref[0]
    fcat = fcat_ref[0]

    pb = pts.astype(bf16).astype(f32)
    tb = ptsT.astype(bf16).astype(f32)
    m = pb[:, 0:1] * tb[0:1, :]
    m = m + pb[:, 1:2] * tb[1:2, :]
    m = m + pb[:, 2:3] * tb[2:3, :]
    r_t = pts[:, 0:1] * pts[:, 0:1]
    r_t = r_t + pts[:, 1:2] * pts[:, 1:2]
    r_t = r_t + pts[:, 2:3] * pts[:, 2:3]
    r_a = ptsT[0:1, :] * ptsT[0:1, :]
    r_a = r_a + ptsT[1:2, :] * ptsT[1:2, :]
    r_a = r_a + ptsT[2:3, :] * ptsT[2:3, :]
    dmat = (r_t - 2.0 * m) + r_a

    ib = jax.lax.bitcast_convert_type(dmat, jnp.int32)
    k1 = jnp.where(ib >= 0, ib, ib ^ jnp.int32(0x7FFFFFFF))
    running = jax.lax.bitcast_convert_type(
        (k1 >> 1) + jnp.int32(0x40000000), f32)
    kmax = f32(3.4028235e38)

    fcb = fc.astype(bf16)
    cpart = jnp.dot(fcb, w0a_ref[...].astype(bf16),
                    preferred_element_type=f32)

    for step in range(k + 1):
        rowmin = jnp.min(running, axis=1, keepdims=True)
        chosen = running == rowmin
        running = jnp.where(chosen, kmax, running)
        if step == 0:
            continue
        oh = chosen.astype(bf16)
        gh = jnp.dot(oh, fcat, preferred_element_type=f32)
        knn = gh[:, :c] + gh[:, c:]
        diff_ref[:, (step - 1) * c:step * c] = (knn - fc).astype(bf16)

    gg = 4 * c
    cpart4 = jnp.concatenate([cpart] * 4, axis=1)
    t4 = lambda ref: jnp.concatenate([ref[...]] * 4, axis=1)
    g0r, b0r = t4(g0_ref), t4(b0_ref)
    g1r, b1r = t4(g1_ref), t4(b1_ref)
    g2r, b2r = t4(g2_ref), t4(b2_ref)
    acc = jnp.zeros((tn, c), f32)
    for g in range(k // 4):
        xg = diff_ref[:, g * gg:(g + 1) * gg]
        y0 = cpart4 + jnp.dot(xg, bd0_ref[...], preferred_element_type=f32)
        h0 = jax.nn.relu(g0r * y0 / sq + b0r)
        y1 = jnp.dot(h0.astype(bf16), bd1_ref[...], preferred_element_type=f32)
        h1 = jax.nn.relu(g1r * y1 / sq + b1r)
        y2 = jnp.dot(h1.astype(bf16), bd2_ref[...], preferred_element_type=f32)
        h2 = jax.nn.relu(g2r * y2 / sq + b2r)
        acc = acc + ((h2[:, :c] + h2[:, c:2 * c])
                     + (h2[:, 2 * c:3 * c] + h2[:, 3 * c:]))

    fts = acc * f32(1.0 / k)
    ysc = jnp.dot(fcb, wsc_ref[...].astype(bf16), preferred_element_type=f32)
    sc = gsc_ref[...] * ysc / sq + bsc_ref[...]
    out_ref[0] = jax.nn.relu(sc + fts)


def kernel(points, features, W0, W1, W2, Wsc, g0, b0, g1, b1, g2, b2, gsc, bsc):
    b, n, c = features.shape
    tn = _TN
    grid = (b, n // tn)

    pointsT = jnp.transpose(points, (0, 2, 1))
    fhi = features.astype(jnp.bfloat16)
    flo = (features - fhi.astype(jnp.float32)).astype(jnp.bfloat16)
    fcat = jnp.concatenate([fhi, flo], axis=2)
    w0a, w0b = W0[:c], W0[c:]
    eye4 = jnp.eye(4, dtype=jnp.float32)
    bd = lambda w: jnp.kron(eye4, w).astype(jnp.bfloat16)
    row = lambda v: v.reshape(1, -1)

    body = functools.partial(_edgeconv_tile, n=n, k=_K)
    out = pl.pallas_call(
        body,
        grid=grid,
        in_specs=[
            pl.BlockSpec((1, tn, points.shape[2]), lambda bi, ti: (bi, ti, 0)),
            pl.BlockSpec((1, points.shape[2], n), lambda bi, ti: (bi, 0, 0)),
            pl.BlockSpec((1, tn, c), lambda bi, ti: (bi, ti, 0)),
            pl.BlockSpec((1, n, 2 * c), lambda bi, ti: (bi, 0, 0)),
            pl.BlockSpec(w0a.shape, lambda bi, ti: (0, 0)),
            pl.BlockSpec((4 * c, 4 * c), lambda bi, ti: (0, 0)),
            pl.BlockSpec((4 * c, 4 * c), lambda bi, ti: (0, 0)),
            pl.BlockSpec((4 * c, 4 * c), lambda bi, ti: (0, 0)),
            pl.BlockSpec(Wsc.shape, lambda bi, ti: (0, 0)),
        ] + [pl.BlockSpec((1, c), lambda bi, ti: (0, 0))] * 8,
        out_specs=pl.BlockSpec((1, tn, c), lambda bi, ti: (bi, ti, 0)),
        out_shape=jax.ShapeDtypeStruct((b, n, c), jnp.float32),
        scratch_shapes=[pltpu.VMEM((tn, _K * c), jnp.bfloat16)],
        compiler_params=pltpu.CompilerParams(
            dimension_semantics=("parallel", "parallel")),
    )(points, pointsT, features, fcat, w0a, bd(w0b), bd(W1), bd(W2), Wsc,
      row(g0), row(b0), row(g1), row(b1), row(g2), row(b2), row(gsc), row(bsc))
    return out
